# baseline (speedup 1.0000x reference)
import functools

import jax
import jax.numpy as jnp
from jax.experimental import pallas as pl

_DIM = 2048
_TOPK = 2
_N_EXPERTS = 8
_ALPHA = 0.0001
_ROUTE_SCALE = 1.0
_NPAD = 128
_BM = 2048
_NWIN = 8


def _gate_kernel(*refs, n_blocks, n_tokens):
    x_refs = refs[:_NWIN]
    w_ref, bias_ref, w_out, i_out, aux_ref, acc_ref = refs[_NWIN:]
    i = pl.program_id(0)

    w = w_ref[...]
    st = jnp.concatenate(
        [jax.lax.dot_general(w, r[...], (((1,), (1,)), ((), ())),
                             preferred_element_type=jnp.float32)
         for r in x_refs],
        axis=1)

    rowf = jax.lax.broadcasted_iota(jnp.int32, (_N_EXPERTS, _BM), 0).astype(
        jnp.float32)
    neg = jnp.float32(-1e30)

    e = jnp.exp(jnp.clip(st, -80.0, 80.0))
    denom = jnp.sum(e, axis=0, keepdims=True)
    p = e / denom

    biased = p + bias_ref[:, 0:1]

    v1 = jnp.max(biased, axis=0, keepdims=True)
    i1 = jnp.min(jnp.where(biased == v1, rowf, jnp.float32(_NPAD)),
                 axis=0, keepdims=True)
    sel1 = rowf == i1
    b2 = jnp.where(sel1, neg, biased)
    v2 = jnp.max(b2, axis=0, keepdims=True)
    i2 = jnp.min(jnp.where(b2 == v2, rowf, jnp.float32(_NPAD)),
                 axis=0, keepdims=True)
    sel2 = rowf == i2

    w1 = jnp.sum(jnp.where(sel1, p, 0.0), axis=0, keepdims=True)
    w2 = jnp.sum(jnp.where(sel2, p, 0.0), axis=0, keepdims=True)

    packed = jnp.concatenate(
        [w1 * _ROUTE_SCALE, w2 * _ROUTE_SCALE, i1, i2,
         jnp.zeros((4, _BM), jnp.float32)], axis=0)
    packed_t = jnp.transpose(packed)
    w_out[...] = packed_t[:, 0:2]
    i_out[...] = packed_t[:, 2:4].astype(jnp.int32)

    part_p = jnp.sum(p, axis=1, keepdims=True)
    part_c = jnp.sum(jnp.where(sel1, 1.0, 0.0) + jnp.where(sel2, 1.0, 0.0),
                     axis=1, keepdims=True)

    @pl.when(i == 0)
    def _init():
        acc_ref[:, 0:1] = part_p
        acc_ref[:, 1:2] = part_c

    @pl.when(i != 0)
    def _acc():
        acc_ref[:, 0:1] = acc_ref[:, 0:1] + part_p
        acc_ref[:, 1:2] = acc_ref[:, 1:2] + part_c

    @pl.when(i == n_blocks - 1)
    def _final():
        scale = jnp.float32(_N_EXPERTS * _ALPHA) / (
            jnp.float32(n_tokens) * jnp.float32(_TOPK * n_tokens))
        aux = jnp.sum(acc_ref[:, 0:1] * acc_ref[:, 1:2], axis=0,
                      keepdims=True) * scale
        aux_ref[...] = aux


@jax.jit
def kernel(x, weight, bias):
    n_tokens = x.shape[0]
    n_blocks = n_tokens // _BM

    bias_col = bias.reshape(_N_EXPERTS, 1)

    grid_spec = pl.GridSpec(
        grid=(n_blocks,),
        in_specs=[
            pl.BlockSpec((_BM // _NWIN, _DIM),
                         lambda i, j=j: (_NWIN * i + j, 0))
            for j in range(_NWIN)
        ] + [
            pl.BlockSpec((_N_EXPERTS, _DIM), lambda i: (0, 0)),
            pl.BlockSpec((_N_EXPERTS, 1), lambda i: (0, 0)),
        ],
        out_specs=[
            pl.BlockSpec((_BM, _TOPK), lambda i: (i, 0)),
            pl.BlockSpec((_BM, _TOPK), lambda i: (i, 0)),
            pl.BlockSpec((1, 1), lambda i: (0, 0)),
            pl.BlockSpec((_N_EXPERTS, _NPAD), lambda i: (0, 0)),
        ],
    )

    weights, indices, aux, _ = pl.pallas_call(
        functools.partial(_gate_kernel, n_blocks=n_blocks, n_tokens=n_tokens),
        grid_spec=grid_spec,
        out_shape=[
            jax.ShapeDtypeStruct((n_tokens, _TOPK), jnp.float32),
            jax.ShapeDtypeStruct((n_tokens, _TOPK), jnp.int32),
            jax.ShapeDtypeStruct((1, 1), jnp.float32),
            jax.ShapeDtypeStruct((_N_EXPERTS, _NPAD), jnp.float32),
        ],
    )(*([x] * _NWIN), weight, bias_col)

    return weights.astype(x.dtype), indices, aux[0, 0]


# speedup vs baseline: 1.0065x; 1.0065x over previous
import functools

import jax
import jax.numpy as jnp
from jax.experimental import pallas as pl

_DIM = 2048
_TOPK = 2
_N_EXPERTS = 8
_ALPHA = 0.0001
_ROUTE_SCALE = 1.0
_NPAD = 128
_BM = 2048


def _gate_kernel(x0_ref, x1_ref, x2_ref, x3_ref, w_ref, bias_ref,
                 w_out, i_out, aux_ref, acc_ref, *, n_blocks, n_tokens):
    i = pl.program_id(0)

    w = w_ref[...]
    st = jnp.concatenate(
        [jax.lax.dot_general(w, r[...], (((1,), (1,)), ((), ())),
                             preferred_element_type=jnp.float32)
         for r in (x0_ref, x1_ref, x2_ref, x3_ref)],
        axis=1)

    rowf = jax.lax.broadcasted_iota(jnp.int32, (_N_EXPERTS, _BM), 0).astype(
        jnp.float32)
    neg = jnp.float32(-1e30)

    e = jnp.exp(jnp.clip(st, -80.0, 80.0))
    denom = jnp.sum(e, axis=0, keepdims=True)
    p = e / denom

    biased = p + bias_ref[:, 0:1]

    v1 = jnp.max(biased, axis=0, keepdims=True)
    i1 = jnp.min(jnp.where(biased == v1, rowf, jnp.float32(_NPAD)),
                 axis=0, keepdims=True)
    sel1 = rowf == i1
    b2 = jnp.where(sel1, neg, biased)
    v2 = jnp.max(b2, axis=0, keepdims=True)
    i2 = jnp.min(jnp.where(b2 == v2, rowf, jnp.float32(_NPAD)),
                 axis=0, keepdims=True)
    sel2 = rowf == i2

    w1 = jnp.sum(jnp.where(sel1, p, 0.0), axis=0, keepdims=True)
    w2 = jnp.sum(jnp.where(sel2, p, 0.0), axis=0, keepdims=True)

    packed = jnp.concatenate(
        [w1 * _ROUTE_SCALE, w2 * _ROUTE_SCALE, i1, i2,
         jnp.zeros((4, _BM), jnp.float32)], axis=0)
    packed_t = jnp.transpose(packed)
    w_out[...] = packed_t[:, 0:2]
    i_out[...] = packed_t[:, 2:4].astype(jnp.int32)

    part_p = jnp.sum(p, axis=1, keepdims=True)
    part_c = jnp.sum(jnp.where(sel1, 1.0, 0.0) + jnp.where(sel2, 1.0, 0.0),
                     axis=1, keepdims=True)

    @pl.when(i == 0)
    def _init():
        acc_ref[:, 0:1] = part_p
        acc_ref[:, 1:2] = part_c

    @pl.when(i != 0)
    def _acc():
        acc_ref[:, 0:1] = acc_ref[:, 0:1] + part_p
        acc_ref[:, 1:2] = acc_ref[:, 1:2] + part_c

    @pl.when(i == n_blocks - 1)
    def _final():
        scale = jnp.float32(_N_EXPERTS * _ALPHA) / (
            jnp.float32(n_tokens) * jnp.float32(_TOPK * n_tokens))
        aux = jnp.sum(acc_ref[:, 0:1] * acc_ref[:, 1:2], axis=0,
                      keepdims=True) * scale
        aux_ref[...] = aux


@jax.jit
def kernel(x, weight, bias):
    n_tokens = x.shape[0]
    n_blocks = n_tokens // _BM

    bias_col = bias.reshape(_N_EXPERTS, 1)

    grid_spec = pl.GridSpec(
        grid=(n_blocks,),
        in_specs=[
            pl.BlockSpec((_BM // 4, _DIM), lambda i: (4 * i + 0, 0)),
            pl.BlockSpec((_BM // 4, _DIM), lambda i: (4 * i + 1, 0)),
            pl.BlockSpec((_BM // 4, _DIM), lambda i: (4 * i + 2, 0)),
            pl.BlockSpec((_BM // 4, _DIM), lambda i: (4 * i + 3, 0)),
            pl.BlockSpec((_N_EXPERTS, _DIM), lambda i: (0, 0)),
            pl.BlockSpec((_N_EXPERTS, 1), lambda i: (0, 0)),
        ],
        out_specs=[
            pl.BlockSpec((_BM, _TOPK), lambda i: (i, 0)),
            pl.BlockSpec((_BM, _TOPK), lambda i: (i, 0)),
            pl.BlockSpec((1, 1), lambda i: (0, 0)),
            pl.BlockSpec((_N_EXPERTS, _NPAD), lambda i: (0, 0)),
        ],
    )

    weights, indices, aux, _ = pl.pallas_call(
        functools.partial(_gate_kernel, n_blocks=n_blocks, n_tokens=n_tokens),
        grid_spec=grid_spec,
        out_shape=[
            jax.ShapeDtypeStruct((n_tokens, _TOPK), jnp.float32),
            jax.ShapeDtypeStruct((n_tokens, _TOPK), jnp.int32),
            jax.ShapeDtypeStruct((1, 1), jnp.float32),
            jax.ShapeDtypeStruct((_N_EXPERTS, _NPAD), jnp.float32),
        ],
    )(x, x, x, x, weight, bias_col)

    return weights.astype(x.dtype), indices, aux[0, 0]
